# baseline (speedup 1.0000x reference)
import functools

import jax
import jax.numpy as jnp
from jax import lax
from jax.experimental import pallas as pl
from jax.experimental.pallas import tpu as pltpu
from jax.experimental.pallas import tpu_sc as plsc

B = 16384
NROW = 1000000
D = 32
BLK = 2048
KC = 128
L = 16


@functools.cache
def _build_gather():
    info = plsc.get_sparse_core_info()
    nc, ns = info.num_cores, info.num_subcores
    nw = nc * ns
    bpw = B // nw
    nk = bpw // KC
    mesh = plsc.VectorSubcoreMesh(core_axis_name="c", subcore_axis_name="s")

    @functools.partial(
        pl.kernel,
        mesh=mesh,
        compiler_params=pltpu.CompilerParams(use_tc_tiling_on_sc=False),
        out_type=[
            jax.ShapeDtypeStruct((D, B), jnp.float32),
            jax.ShapeDtypeStruct((D, B), jnp.float32),
        ],
        scratch_types=[
            pltpu.VMEM((bpw,), jnp.int32),
            pltpu.VMEM((bpw,), jnp.int32),
            pltpu.VMEM((D, bpw), jnp.int32),
            pltpu.VMEM((D, bpw), jnp.int32),
            pltpu.VMEM((D, bpw), jnp.float32),
            pltpu.VMEM((D, bpw), jnp.float32),
            pltpu.SemaphoreType.DMA,
            pltpu.SemaphoreType.DMA,
        ],
    )
    def gather(u_hbm, v_hbm, uf_hbm, vf_hbm, euT_hbm, evT_hbm,
               ui_v, vi_v, uo_v, vo_v, cu_v, cv_v, sem_u, sem_v):
        wid = lax.axis_index("s") * nc + lax.axis_index("c")
        base = wid * bpw
        pltpu.sync_copy(u_hbm.at[pl.ds(base, bpw)], ui_v)
        pltpu.sync_copy(v_hbm.at[pl.ds(base, bpw)], vi_v)

        def build_offsets(c, carry):
            off = c * NROW
            for g in range(bpw // L):
                sl = pl.ds(g * L, L)
                uo_v[c, sl] = ui_v[sl] + off
                vo_v[c, sl] = vi_v[sl] + off
            return carry

        lax.fori_loop(0, D, build_offsets, 0)

        def issue(c):
            for k in range(nk):
                sl = pl.ds(k * KC, KC)
                pltpu.async_copy(
                    uf_hbm.at[uo_v.at[c, sl]], cu_v.at[c, sl], sem_u)
                pltpu.async_copy(
                    vf_hbm.at[vo_v.at[c, sl]], cv_v.at[c, sl], sem_v)

        def drain():
            pltpu.make_async_copy(
                uf_hbm.at[pl.ds(0, bpw)], cu_v.at[0], sem_u).wait()
            pltpu.make_async_copy(
                vf_hbm.at[pl.ds(0, bpw)], cv_v.at[0], sem_v).wait()

        issue(0)

        def body(c, carry):
            issue(c)
            drain()
            return carry

        lax.fori_loop(1, D, body, 0)
        drain()

        pltpu.sync_copy(cu_v, euT_hbm.at[:, pl.ds(base, bpw)])
        pltpu.sync_copy(cv_v, evT_hbm.at[:, pl.ds(base, bpw)])

    return gather


def _mlp_body(euT_ref, evT_ref, w1u_ref, w1v_ref, b1_ref, w2_ref, b2_ref,
              whg_ref, whh_ref, bh_ref, out_ref):
    euT = euT_ref[...]
    evT = evT_ref[...]
    h1 = jnp.dot(w1u_ref[...], euT, preferred_element_type=jnp.float32)
    h1 = h1 + jnp.dot(w1v_ref[...], evT, preferred_element_type=jnp.float32)
    h1 = jnp.maximum(h1 + b1_ref[...], 0.0)
    h2 = jnp.dot(w2_ref[...], h1, preferred_element_type=jnp.float32)
    h2 = jnp.maximum(h2 + b2_ref[...], 0.0)
    gmfT = euT * evT
    logit = jnp.dot(whg_ref[...], gmfT, preferred_element_type=jnp.float32)
    logit = logit + jnp.dot(whh_ref[...], h2, preferred_element_type=jnp.float32)
    logit = logit + bh_ref[...]
    out_ref[...] = jax.nn.sigmoid(logit)


def _mlp_call(euT, evT, w1u, w1v, b1, w2, b2, whg, whh, bh2d):
    grid = B // BLK
    full = lambda i: (0, 0)
    return pl.pallas_call(
        _mlp_body,
        grid=(grid,),
        in_specs=[
            pl.BlockSpec((D, BLK), lambda i: (0, i)),
            pl.BlockSpec((D, BLK), lambda i: (0, i)),
            pl.BlockSpec((128, D), full),
            pl.BlockSpec((128, D), full),
            pl.BlockSpec((128, 1), full),
            pl.BlockSpec((32, 128), full),
            pl.BlockSpec((32, 1), full),
            pl.BlockSpec((1, D), full),
            pl.BlockSpec((1, 32), full),
            pl.BlockSpec((1, 1), full),
        ],
        out_specs=pl.BlockSpec((1, BLK), lambda i: (0, i)),
        out_shape=jax.ShapeDtypeStruct((1, B), jnp.float32),
    )(euT, evT, w1u, w1v, b1, w2, b2, whg, whh, bh2d)


def kernel(u, v, U_gmf, V_gmf, W1, b1, W2, b2, Wh, bh):
    uf = jnp.reshape(U_gmf.T, (-1,))
    vf = jnp.reshape(V_gmf.T, (-1,))
    euT, evT = _build_gather()(u.astype(jnp.int32), v.astype(jnp.int32),
                               uf, vf)
    w1u = W1[:, :D]
    w1v = W1[:, D:]
    whg = Wh[:, :D]
    whh = Wh[:, D:]
    out = _mlp_call(euT, evT, w1u, w1v, b1.reshape(128, 1), W2,
                    b2.reshape(32, 1), whg, whh, bh.reshape(1, 1))
    return out.reshape(B, 1)


# speedup vs baseline: 1.0039x; 1.0039x over previous
import functools

import jax
import jax.numpy as jnp
from jax import lax
from jax.experimental import pallas as pl
from jax.experimental.pallas import tpu as pltpu
from jax.experimental.pallas import tpu_sc as plsc

B = 16384
NROW = 1000000
D = 32
BLK = 2048
KC = 128
L = 16


@functools.cache
def _build_gather():
    info = plsc.get_sparse_core_info()
    nc, ns = info.num_cores, info.num_subcores
    nw = nc * ns
    bpw = B // nw
    nk = bpw // KC
    mesh = plsc.VectorSubcoreMesh(core_axis_name="c", subcore_axis_name="s")

    @functools.partial(
        pl.kernel,
        mesh=mesh,
        compiler_params=pltpu.CompilerParams(use_tc_tiling_on_sc=False),
        out_type=[
            jax.ShapeDtypeStruct((D, B), jnp.float32),
            jax.ShapeDtypeStruct((D, B), jnp.float32),
        ],
        scratch_types=[
            pltpu.VMEM((bpw,), jnp.int32),
            pltpu.VMEM((bpw,), jnp.int32),
            pltpu.VMEM((D, bpw), jnp.int32),
            pltpu.VMEM((D, bpw), jnp.int32),
            pltpu.VMEM((D, bpw), jnp.float32),
            pltpu.VMEM((D, bpw), jnp.float32),
            pltpu.SemaphoreType.DMA,
            pltpu.SemaphoreType.DMA,
        ],
    )
    def gather(u_hbm, v_hbm, uf_hbm, vf_hbm, euT_hbm, evT_hbm,
               ui_v, vi_v, uo_v, vo_v, cu_v, cv_v, sem_u, sem_v):
        wid = lax.axis_index("s") * nc + lax.axis_index("c")
        base = wid * bpw
        pltpu.sync_copy(u_hbm.at[pl.ds(base, bpw)], ui_v)
        pltpu.sync_copy(v_hbm.at[pl.ds(base, bpw)], vi_v)

        def build_offsets(c, carry):
            off = c * NROW
            for g in range(bpw // L):
                sl = pl.ds(g * L, L)
                uo_v[c, sl] = ui_v[sl] + off
                vo_v[c, sl] = vi_v[sl] + off
            return carry

        lax.fori_loop(0, D, build_offsets, 0)

        def issue(c):
            for k in range(nk):
                sl = pl.ds(k * KC, KC)
                pltpu.async_copy(
                    uf_hbm.at[uo_v.at[c, sl]], cu_v.at[c, sl], sem_u)
                pltpu.async_copy(
                    vf_hbm.at[vo_v.at[c, sl]], cv_v.at[c, sl], sem_v)

        def drain():
            pltpu.make_async_copy(
                uf_hbm.at[pl.ds(0, bpw)], cu_v.at[0], sem_u).wait()
            pltpu.make_async_copy(
                vf_hbm.at[pl.ds(0, bpw)], cv_v.at[0], sem_v).wait()

        issue(0)

        def body(c, carry):
            issue(c)
            drain()
            return carry

        lax.fori_loop(1, D, body, 0)
        drain()

        pltpu.sync_copy(cu_v, euT_hbm.at[:, pl.ds(base, bpw)])
        pltpu.sync_copy(cv_v, evT_hbm.at[:, pl.ds(base, bpw)])

    return gather


def _mlp_body(euT_ref, evT_ref, w1u_ref, w1v_ref, b1_ref, w2_ref, b2_ref,
              whg_ref, whh_ref, bh_ref, out_ref):
    euT = euT_ref[...]
    evT = evT_ref[...]
    h1 = jnp.dot(w1u_ref[...], euT, preferred_element_type=jnp.float32)
    h1 = h1 + jnp.dot(w1v_ref[...], evT, preferred_element_type=jnp.float32)
    h1 = jnp.maximum(h1 + b1_ref[...], 0.0)
    h2 = jnp.dot(w2_ref[...], h1, preferred_element_type=jnp.float32)
    h2 = jnp.maximum(h2 + b2_ref[...], 0.0)
    gmfT = euT * evT
    logit = jnp.dot(whg_ref[...], gmfT, preferred_element_type=jnp.float32)
    logit = logit + jnp.dot(whh_ref[...], h2, preferred_element_type=jnp.float32)
    logit = logit + bh_ref[...]
    out_ref[...] = jax.nn.sigmoid(logit)


def _mlp_call(euT, evT, w1u, w1v, b1, w2, b2, whg, whh, bh2d):
    grid = B // BLK
    full = lambda i: (0, 0)
    return pl.pallas_call(
        _mlp_body,
        grid=(grid,),
        in_specs=[
            pl.BlockSpec((D, BLK), lambda i: (0, i)),
            pl.BlockSpec((D, BLK), lambda i: (0, i)),
            pl.BlockSpec((128, D), full),
            pl.BlockSpec((128, D), full),
            pl.BlockSpec((128, 1), full),
            pl.BlockSpec((32, 128), full),
            pl.BlockSpec((32, 1), full),
            pl.BlockSpec((1, D), full),
            pl.BlockSpec((1, 32), full),
            pl.BlockSpec((1, 1), full),
        ],
        out_specs=pl.BlockSpec((1, BLK), lambda i: (0, i)),
        out_shape=jax.ShapeDtypeStruct((1, B), jnp.float32),
    )(euT, evT, w1u, w1v, b1, w2, b2, whg, whh, bh2d)


def kernel(u, v, U_gmf, V_gmf, W1, b1, W2, b2, Wh, bh):
    ut, vt = jax.lax.optimization_barrier((U_gmf.T, V_gmf.T))
    uf = jnp.reshape(ut, (-1,))
    vf = jnp.reshape(vt, (-1,))
    euT, evT = _build_gather()(u.astype(jnp.int32), v.astype(jnp.int32),
                               uf, vf)
    w1u = W1[:, :D]
    w1v = W1[:, D:]
    whg = Wh[:, :D]
    whh = Wh[:, D:]
    out = _mlp_call(euT, evT, w1u, w1v, b1.reshape(128, 1), W2,
                    b2.reshape(32, 1), whg, whh, bh.reshape(1, 1))
    return out.reshape(B, 1)


# speedup vs baseline: 5.5554x; 5.5338x over previous
import functools

import jax
import jax.numpy as jnp
from jax import lax
from jax.experimental import pallas as pl
from jax.experimental.pallas import tpu as pltpu
from jax.experimental.pallas import tpu_sc as plsc

B = 16384
D = 32
BLK = 2048


@functools.cache
def _build_gather():
    info = plsc.get_sparse_core_info()
    nc, ns = info.num_cores, info.num_subcores
    nw = nc * ns
    bpw = B // nw
    mesh = plsc.VectorSubcoreMesh(core_axis_name="c", subcore_axis_name="s")

    @functools.partial(
        pl.kernel,
        mesh=mesh,
        compiler_params=pltpu.CompilerParams(use_tc_tiling_on_sc=False),
        out_type=[
            jax.ShapeDtypeStruct((B, D), jnp.float32),
            jax.ShapeDtypeStruct((B, D), jnp.float32),
        ],
        scratch_types=[
            pltpu.VMEM((bpw,), jnp.int32),
            pltpu.VMEM((bpw,), jnp.int32),
            pltpu.VMEM((bpw, D), jnp.float32),
            pltpu.VMEM((bpw, D), jnp.float32),
            pltpu.SemaphoreType.DMA,
            pltpu.SemaphoreType.DMA,
        ],
    )
    def gather(u_hbm, v_hbm, ut_hbm, vt_hbm, eu_hbm, ev_hbm,
               ui_v, vi_v, eu_v, ev_v, sem_u, sem_v):
        wid = lax.axis_index("s") * nc + lax.axis_index("c")
        base = wid * bpw
        pltpu.sync_copy(u_hbm.at[pl.ds(base, bpw)], ui_v)
        pltpu.sync_copy(v_hbm.at[pl.ds(base, bpw)], vi_v)
        cp_u = pltpu.async_copy(ut_hbm.at[ui_v], eu_v, sem_u)
        cp_v = pltpu.async_copy(vt_hbm.at[vi_v], ev_v, sem_v)
        cp_u.wait()
        cp_v.wait()
        pltpu.sync_copy(eu_v, eu_hbm.at[pl.ds(base, bpw)])
        pltpu.sync_copy(ev_v, ev_hbm.at[pl.ds(base, bpw)])

    return gather


def _mlp_body(eu_ref, ev_ref, w1u_ref, w1v_ref, b1_ref, w2t_ref, b2_ref,
              whg_ref, whh_ref, bh_ref, out_ref):
    eu = eu_ref[...]
    ev = ev_ref[...]
    h1 = jnp.dot(eu, w1u_ref[...], preferred_element_type=jnp.float32)
    h1 = h1 + jnp.dot(ev, w1v_ref[...], preferred_element_type=jnp.float32)
    h1 = jnp.maximum(h1 + b1_ref[...], 0.0)
    h2 = jnp.dot(h1, w2t_ref[...], preferred_element_type=jnp.float32)
    h2 = jnp.maximum(h2 + b2_ref[...], 0.0)
    gmf = eu * ev
    logit = jnp.dot(gmf, whg_ref[...], preferred_element_type=jnp.float32)
    logit = logit + jnp.dot(h2, whh_ref[...], preferred_element_type=jnp.float32)
    logit = logit + bh_ref[...]
    out_ref[...] = jax.nn.sigmoid(logit)


def _mlp_call(eu, ev, w1u, w1v, b1, w2t, b2, whg, whh, bh2d):
    grid = B // BLK
    full = lambda i: (0, 0)
    return pl.pallas_call(
        _mlp_body,
        grid=(grid,),
        in_specs=[
            pl.BlockSpec((BLK, D), lambda i: (i, 0)),
            pl.BlockSpec((BLK, D), lambda i: (i, 0)),
            pl.BlockSpec((D, 128), full),
            pl.BlockSpec((D, 128), full),
            pl.BlockSpec((1, 128), full),
            pl.BlockSpec((128, 32), full),
            pl.BlockSpec((1, 32), full),
            pl.BlockSpec((D, 1), full),
            pl.BlockSpec((32, 1), full),
            pl.BlockSpec((1, 1), full),
        ],
        out_specs=pl.BlockSpec((BLK, 1), lambda i: (i, 0)),
        out_shape=jax.ShapeDtypeStruct((B, 1), jnp.float32),
    )(eu, ev, w1u, w1v, b1, w2t, b2, whg, whh, bh2d)


def kernel(u, v, U_gmf, V_gmf, W1, b1, W2, b2, Wh, bh):
    eu, ev = _build_gather()(u.astype(jnp.int32), v.astype(jnp.int32),
                             U_gmf, V_gmf)
    w1u = W1[:, :D].T
    w1v = W1[:, D:].T
    w2t = W2.T
    whg = Wh[0, :D].reshape(D, 1)
    whh = Wh[0, D:].reshape(32, 1)
    return _mlp_call(eu, ev, w1u, w1v, b1.reshape(1, 128), w2t,
                     b2.reshape(1, 32), whg, whh, bh.reshape(1, 1))
